# baseline (speedup 1.0000x reference)
import functools

import jax
import jax.numpy as jnp
from jax import lax
from jax.experimental import pallas as pl
from jax.experimental.pallas import tpu as pltpu
from jax.experimental.pallas import tpu_sc as plsc

_NC = 2
_NS = 16
_NW = _NC * _NS

def _sc_mesh():
    return plsc.VectorSubcoreMesh(
        core_axis_name="c", subcore_axis_name="s",
        num_cores=_NC, num_subcores=_NS,
    )


def _layernorm_affine(x, g, b):
    mu = jnp.mean(x, axis=-1, keepdims=True)
    var = jnp.mean((x - mu) ** 2, axis=-1, keepdims=True)
    return (x - mu) * lax.rsqrt(var + 1e-5) * g + b


def _proj_kernel(x_ref, ws_ref, wr_ref, ps_ref, pr_ref):
    x = x_ref[...]
    ps_ref[...] = jnp.dot(x, ws_ref[...], preferred_element_type=jnp.float32)
    pr_ref[...] = jnp.dot(x, wr_ref[...], preferred_element_type=jnp.float32)


def _proj(h_nodes, w_s, w_r):
    n, d = h_nodes.shape
    h = w_s.shape[1]
    return pl.pallas_call(
        _proj_kernel,
        out_shape=[
            jax.ShapeDtypeStruct((n, h), jnp.float32),
            jax.ShapeDtypeStruct((n, h), jnp.float32),
        ],
    )(h_nodes, w_s, w_r)


def _gather_body(ps_hbm, pr_hbm, snd_hbm, rcv_hbm, gs_hbm, gr_hbm,
                 ids_v, idr_v, rs0, rr0, rs1, rr1,
                 gs0, gr0, gs1, gr1, ws0, wr0, ws1, wr1,
                 *, epw, chunk, ebase):
    wid = lax.axis_index("s") * _NC + lax.axis_index("c")
    base = wid * epw
    npairs = (epw // chunk) // 2

    pltpu.sync_copy(snd_hbm.at[pl.ds(ebase + base, epw)], ids_v)
    pltpu.sync_copy(rcv_hbm.at[pl.ds(ebase + base, epw)], idr_v)

    def g_start(i, rs, rr, sem_s, sem_r):
        pltpu.async_copy(ps_hbm.at[ids_v.at[pl.ds(i * chunk, chunk)]], rs,
                         sem_s)
        pltpu.async_copy(pr_hbm.at[idr_v.at[pl.ds(i * chunk, chunk)]], rr,
                         sem_r)

    def g_wait(rs, rr, sem_s, sem_r):
        pltpu.make_async_copy(ps_hbm.at[ids_v.at[pl.ds(0, chunk)]], rs,
                              sem_s).wait()
        pltpu.make_async_copy(pr_hbm.at[idr_v.at[pl.ds(0, chunk)]], rr,
                              sem_r).wait()

    def w_start(i, rs, rr, sem_s, sem_r):
        off = base + i * chunk
        pltpu.async_copy(rs, gs_hbm.at[pl.ds(off, chunk)], sem_s)
        pltpu.async_copy(rr, gr_hbm.at[pl.ds(off, chunk)], sem_r)

    def w_wait(rs, rr, sem_s, sem_r):
        pltpu.make_async_copy(rs, gs_hbm.at[pl.ds(base, chunk)], sem_s).wait()
        pltpu.make_async_copy(rr, gr_hbm.at[pl.ds(base, chunk)], sem_r).wait()

    g_start(0, rs0, rr0, gs0, gr0)

    def pair(j, carry):
        i0 = 2 * j
        g_wait(rs0, rr0, gs0, gr0)
        w_start(i0, rs0, rr0, ws0, wr0)

        @pl.when(j > 0)
        def _():
            w_wait(rs1, rr1, ws1, wr1)

        g_start(i0 + 1, rs1, rr1, gs1, gr1)
        g_wait(rs1, rr1, gs1, gr1)
        w_start(i0 + 1, rs1, rr1, ws1, wr1)
        w_wait(rs0, rr0, ws0, wr0)

        @pl.when(j + 1 < npairs)
        def _():
            g_start(i0 + 2, rs0, rr0, gs0, gr0)

        return carry

    lax.fori_loop(0, npairs, pair, 0, unroll=False)
    w_wait(rs1, rr1, ws1, wr1)


def _gather(ps, pr, sender, receiver, ebase, es):
    n, h = ps.shape
    epw = es // _NW
    chunk = 200
    kern = functools.partial(_gather_body, epw=epw, chunk=chunk, ebase=ebase)
    return pl.kernel(
        kern,
        mesh=_sc_mesh(),
        out_type=[
            jax.ShapeDtypeStruct((es, h), jnp.float32),
            jax.ShapeDtypeStruct((es, h), jnp.float32),
        ],
        scratch_types=[
            pltpu.VMEM((epw,), jnp.int32),
            pltpu.VMEM((epw,), jnp.int32),
            pltpu.VMEM((chunk, h), jnp.float32),
            pltpu.VMEM((chunk, h), jnp.float32),
            pltpu.VMEM((chunk, h), jnp.float32),
            pltpu.VMEM((chunk, h), jnp.float32),
        ] + [pltpu.SemaphoreType.DMA] * 8,
    )(ps, pr, sender, receiver)


def _edge_mlp_whole_kernel(*refs):
    _edge_mlp_kernel(None, *refs)


def _edge_mlp_kernel(dst_ref, he_ref, gs_ref, gr_ref, w1_ref, b1_ref, w2_ref,
                     b2_ref, w3_ref, b3_ref, g_ref, bg_ref, upd_ref, oute_ref):
    del dst_ref
    he = he_ref[...]
    x = jnp.dot(he, w1_ref[...], preferred_element_type=jnp.float32)
    x = x + gs_ref[...] + gr_ref[...] + b1_ref[...]
    x = jnp.maximum(x, 0.0)
    x = jnp.dot(x, w2_ref[...], preferred_element_type=jnp.float32) + b2_ref[...]
    x = jnp.maximum(x, 0.0)
    x = jnp.dot(x, w3_ref[...], preferred_element_type=jnp.float32) + b3_ref[...]
    u = _layernorm_affine(x, g_ref[...], bg_ref[...])
    upd_ref[...] = u
    oute_ref[...] = he + u


def _edge_mlp(oute_in, h_edges, gs, gr, w1e, b1, w2, b2, w3, b3, g, bg,
              ebase, es):
    e, d = h_edges.shape
    h = w1e.shape[1]
    be = min(4000, es)
    grid = (es // be,)
    boff = ebase // be
    whole = es == e
    slice_spec = pl.BlockSpec((be, d), lambda i: (boff + i, 0))
    local_spec = pl.BlockSpec((be, d), lambda i: (i, 0))
    full = lambda a, b: pl.BlockSpec((a, b), lambda i: (0, 0))
    dst_in_specs = [] if whole else [
        pl.BlockSpec(memory_space=pltpu.MemorySpace.HBM)]
    dst_args = () if whole else (oute_in,)
    body = _edge_mlp_whole_kernel if whole else _edge_mlp_kernel
    return pl.pallas_call(
        body,
        grid=grid,
        in_specs=dst_in_specs + [
            slice_spec, local_spec, local_spec,
            full(d, h), full(1, h), full(h, h), full(1, h),
            full(h, d), full(1, d), full(1, d), full(1, d),
        ],
        out_specs=[local_spec, slice_spec],
        out_shape=[
            jax.ShapeDtypeStruct((es, d), jnp.float32),
            jax.ShapeDtypeStruct((e, d), jnp.float32),
        ],
        input_output_aliases={} if whole else {0: 1},
        compiler_params=pltpu.CompilerParams(
            dimension_semantics=("parallel",),
        ),
    )(*dst_args, h_edges, gs, gr, w1e, b1.reshape(1, -1), w2,
      b2.reshape(1, -1), w3, b3.reshape(1, -1), g.reshape(1, -1),
      bg.reshape(1, -1))


def _scatter_body(upd_hbm, rcv_hbm, zeros_hbm, out_hbm,
                  idx2d_v, rows0, rows1, sem0, sem1, acc_sh,
                  *, epw, chunk, nps, rem):
    cid = lax.axis_index("c")
    sid = lax.axis_index("s")
    wid = sid * _NC + cid
    base = wid * epw
    niters = epw // chunk
    npairs = niters // 2

    def _sliced_copy(src, dst):
        pltpu.sync_copy(src.at[pl.ds(sid * nps, nps)],
                        dst.at[pl.ds(sid * nps, nps)])
        if rem:
            @pl.when(sid == _NS - 1)
            def _():
                pltpu.sync_copy(src.at[pl.ds(_NS * nps, rem)],
                                dst.at[pl.ds(_NS * nps, rem)])

    _sliced_copy(zeros_hbm, acc_sh)
    pltpu.sync_copy(rcv_hbm.at[wid], idx2d_v)
    plsc.subcore_barrier()

    def l_start(i, rows, sem):
        pltpu.async_copy(upd_hbm.at[pl.ds(base + i * chunk, chunk)], rows,
                         sem)

    def l_wait(rows, sem):
        pltpu.make_async_copy(upd_hbm.at[pl.ds(base, chunk)], rows,
                              sem).wait()

    def add(i, rows):
        pltpu.sync_copy(rows, acc_sh.at[idx2d_v.at[i]], add=True)

    l_start(0, rows0, sem0)

    def pair(j, carry):
        i0 = 2 * j
        l_wait(rows0, sem0)
        l_start(i0 + 1, rows1, sem1)
        add(i0, rows0)
        l_wait(rows1, sem1)

        @pl.when(j + 1 < npairs)
        def _():
            l_start(i0 + 2, rows0, sem0)

        add(i0 + 1, rows1)
        return carry

    lax.fori_loop(0, npairs, pair, 0, unroll=False)
    plsc.subcore_barrier()
    _sliced_copy(acc_sh, out_hbm.at[cid])


def _scatter(upd, receiver, n, ebase):
    e, h = upd.shape
    epw = e // _NW
    chunk = 40
    niters = epw // chunk
    nps = (n // _NS) // 8 * 8
    rem = n - _NS * nps
    zeros = jnp.zeros((n, h), jnp.float32)
    rcv3d = lax.dynamic_slice(receiver, (ebase,), (e,)).reshape(
        _NW, niters, chunk)
    kern = functools.partial(_scatter_body, epw=epw, chunk=chunk, nps=nps,
                             rem=rem)
    return pl.kernel(
        kern,
        mesh=_sc_mesh(),
        out_type=jax.ShapeDtypeStruct((_NC, n, h), jnp.float32),
        scratch_types=[
            pltpu.VMEM((niters, chunk), jnp.int32),
            pltpu.VMEM((chunk, h), jnp.float32),
            pltpu.VMEM((chunk, h), jnp.float32),
            pltpu.SemaphoreType.DMA,
            pltpu.SemaphoreType.DMA,
            pltpu.VMEM_SHARED((n, h), jnp.float32),
        ],
    )(upd, rcv3d, zeros)


def _node_mlp_kernel(*refs, nparts):
    hn_ref = refs[0]
    part_refs = refs[1:1 + nparts]
    (w1t_ref, w1b_ref, b1_ref, w2_ref, b2_ref, w3_ref, b3_ref, g_ref,
     bg_ref, out_ref) = refs[1 + nparts:]
    hn = hn_ref[...]
    agg = part_refs[0][...]
    for p_ref in part_refs[1:]:
        agg = agg + p_ref[...]
    x = (jnp.dot(hn, w1t_ref[...], preferred_element_type=jnp.float32)
         + jnp.dot(agg, w1b_ref[...], preferred_element_type=jnp.float32)
         + b1_ref[...])
    x = jnp.maximum(x, 0.0)
    x = jnp.dot(x, w2_ref[...], preferred_element_type=jnp.float32) + b2_ref[...]
    x = jnp.maximum(x, 0.0)
    x = jnp.dot(x, w3_ref[...], preferred_element_type=jnp.float32) + b3_ref[...]
    u = _layernorm_affine(x, g_ref[...], bg_ref[...])
    out_ref[...] = hn + u


def _node_mlp(h_nodes, parts, w1t, w1b, b1, w2, b2, w3, b3, g, bg):
    n, d = h_nodes.shape
    h = w1t.shape[1]
    bn = min(2000, n)
    grid = (n // bn,)
    nparts = len(parts)
    row_spec = pl.BlockSpec((bn, d), lambda i: (i, 0))
    full = lambda a, b: pl.BlockSpec((a, b), lambda i: (0, 0))
    return pl.pallas_call(
        functools.partial(_node_mlp_kernel, nparts=nparts),
        grid=grid,
        in_specs=[
            row_spec, *([row_spec] * nparts),
            full(d, h), full(d, h), full(1, h), full(h, h), full(1, h),
            full(h, d), full(1, d), full(1, d), full(1, d),
        ],
        out_specs=row_spec,
        out_shape=jax.ShapeDtypeStruct((n, d), jnp.float32),
        compiler_params=pltpu.CompilerParams(
            dimension_semantics=("parallel",),
        ),
    )(h_nodes, *parts, w1t, w1b, b1.reshape(1, -1),
      w2, b2.reshape(1, -1), w3, b3.reshape(1, -1),
      g.reshape(1, -1), bg.reshape(1, -1))


def kernel(h_nodes, h_edges, edge_index, We1, be1, We2, be2, We3, be3, ge,
           bge, Wn1, bn1, Wn2, bn2, Wn3, bn3, gn, bgn):
    n, d = h_nodes.shape
    e = h_edges.shape[0]
    sender = edge_index[0]
    receiver = edge_index[1]
    w1e, w1s, w1r = We1[:d], We1[d:2 * d], We1[2 * d:]

    nslices = 1
    es = e // nslices

    ps, pr = _proj(h_nodes, w1s, w1r)
    gathered = [_gather(ps, pr, sender, receiver, k * es, es)
                for k in range(nslices)]
    out_edges = None if nslices == 1 else jnp.zeros((e, d), jnp.float32)
    parts = []
    for k, (gs, gr) in enumerate(gathered):
        upd, out_edges = _edge_mlp(out_edges, h_edges, gs, gr, w1e, be1, We2,
                                   be2, We3, be3, ge, bge, k * es, es)
        pk = _scatter(upd, receiver, n, k * es)
        parts.extend([pk[0], pk[1]])
    out_nodes = _node_mlp(h_nodes, parts, Wn1[:d], Wn1[d:], bn1, Wn2, bn2,
                          Wn3, bn3, gn, bgn)
    return out_nodes, out_edges


# speedup vs baseline: 1.1096x; 1.1096x over previous
import functools

import jax
import jax.numpy as jnp
from jax import lax
from jax.experimental import pallas as pl
from jax.experimental.pallas import tpu as pltpu
from jax.experimental.pallas import tpu_sc as plsc

_NC = 2
_NS = 16
_NW = _NC * _NS

def _sc_mesh():
    return plsc.VectorSubcoreMesh(
        core_axis_name="c", subcore_axis_name="s",
        num_cores=_NC, num_subcores=_NS,
    )


def _layernorm_affine(x, g, b):
    mu = jnp.mean(x, axis=-1, keepdims=True)
    var = jnp.mean((x - mu) ** 2, axis=-1, keepdims=True)
    return (x - mu) * lax.rsqrt(var + 1e-5) * g + b


def _proj_kernel(x_ref, ws_ref, wr_ref, ps_ref, pr_ref):
    x = x_ref[...]
    ps_ref[...] = jnp.dot(x, ws_ref[...], preferred_element_type=jnp.float32)
    pr_ref[...] = jnp.dot(x, wr_ref[...], preferred_element_type=jnp.float32)


def _proj(h_nodes, w_s, w_r):
    n, d = h_nodes.shape
    h = w_s.shape[1]
    return pl.pallas_call(
        _proj_kernel,
        out_shape=[
            jax.ShapeDtypeStruct((n, h), jnp.float32),
            jax.ShapeDtypeStruct((n, h), jnp.float32),
        ],
    )(h_nodes, w_s, w_r)


def _gather_body(ps_hbm, pr_hbm, snd_hbm, rcv_hbm, gs_hbm, gr_hbm,
                 ids_v, idr_v, rs0, rr0, rs1, rr1,
                 gs0, gr0, gs1, gr1, ws0, wr0, ws1, wr1,
                 *, epw, chunk, ebase):
    wid = lax.axis_index("s") * _NC + lax.axis_index("c")
    base = wid * epw
    npairs = (epw // chunk) // 2

    pltpu.sync_copy(snd_hbm.at[pl.ds(ebase + base, epw)], ids_v)
    pltpu.sync_copy(rcv_hbm.at[pl.ds(ebase + base, epw)], idr_v)

    def g_start(i, rs, rr, sem_s, sem_r):
        pltpu.async_copy(ps_hbm.at[ids_v.at[pl.ds(i * chunk, chunk)]], rs,
                         sem_s)
        pltpu.async_copy(pr_hbm.at[idr_v.at[pl.ds(i * chunk, chunk)]], rr,
                         sem_r)

    def g_wait(rs, rr, sem_s, sem_r):
        pltpu.make_async_copy(ps_hbm.at[ids_v.at[pl.ds(0, chunk)]], rs,
                              sem_s).wait()
        pltpu.make_async_copy(pr_hbm.at[idr_v.at[pl.ds(0, chunk)]], rr,
                              sem_r).wait()

    def w_start(i, rs, rr, sem_s, sem_r):
        off = base + i * chunk
        pltpu.async_copy(rs, gs_hbm.at[pl.ds(off, chunk)], sem_s)
        pltpu.async_copy(rr, gr_hbm.at[pl.ds(off, chunk)], sem_r)

    def w_wait(rs, rr, sem_s, sem_r):
        pltpu.make_async_copy(rs, gs_hbm.at[pl.ds(base, chunk)], sem_s).wait()
        pltpu.make_async_copy(rr, gr_hbm.at[pl.ds(base, chunk)], sem_r).wait()

    g_start(0, rs0, rr0, gs0, gr0)

    def pair(j, carry):
        i0 = 2 * j
        g_wait(rs0, rr0, gs0, gr0)
        w_start(i0, rs0, rr0, ws0, wr0)

        @pl.when(j > 0)
        def _():
            w_wait(rs1, rr1, ws1, wr1)

        g_start(i0 + 1, rs1, rr1, gs1, gr1)
        g_wait(rs1, rr1, gs1, gr1)
        w_start(i0 + 1, rs1, rr1, ws1, wr1)
        w_wait(rs0, rr0, ws0, wr0)

        @pl.when(j + 1 < npairs)
        def _():
            g_start(i0 + 2, rs0, rr0, gs0, gr0)

        return carry

    lax.fori_loop(0, npairs, pair, 0, unroll=False)
    w_wait(rs1, rr1, ws1, wr1)


def _gather(ps, pr, sender, receiver, ebase, es):
    n, h = ps.shape
    epw = es // _NW
    chunk = 200
    kern = functools.partial(_gather_body, epw=epw, chunk=chunk, ebase=ebase)
    return pl.kernel(
        kern,
        mesh=_sc_mesh(),
        out_type=[
            jax.ShapeDtypeStruct((es, h), jnp.float32),
            jax.ShapeDtypeStruct((es, h), jnp.float32),
        ],
        scratch_types=[
            pltpu.VMEM((epw,), jnp.int32),
            pltpu.VMEM((epw,), jnp.int32),
            pltpu.VMEM((chunk, h), jnp.float32),
            pltpu.VMEM((chunk, h), jnp.float32),
            pltpu.VMEM((chunk, h), jnp.float32),
            pltpu.VMEM((chunk, h), jnp.float32),
        ] + [pltpu.SemaphoreType.DMA] * 8,
    )(ps, pr, sender, receiver)


def _edge_mlp_whole_kernel(*refs):
    _edge_mlp_kernel(None, *refs)


def _edge_mlp_kernel(dst_ref, he_ref, gs_ref, gr_ref, w1_ref, b1_ref, w2_ref,
                     b2_ref, w3_ref, b3_ref, g_ref, bg_ref, upd_ref, oute_ref):
    del dst_ref
    he = he_ref[...]
    x = jnp.dot(he, w1_ref[...], preferred_element_type=jnp.float32)
    x = x + gs_ref[...] + gr_ref[...] + b1_ref[...]
    x = jnp.maximum(x, 0.0)
    x = jnp.dot(x, w2_ref[...], preferred_element_type=jnp.float32) + b2_ref[...]
    x = jnp.maximum(x, 0.0)
    x = jnp.dot(x, w3_ref[...], preferred_element_type=jnp.float32) + b3_ref[...]
    u = _layernorm_affine(x, g_ref[...], bg_ref[...])
    upd_ref[...] = u
    oute_ref[...] = he + u


def _edge_mlp(oute_in, h_edges, gs, gr, w1e, b1, w2, b2, w3, b3, g, bg,
              ebase, es):
    e, d = h_edges.shape
    h = w1e.shape[1]
    be = min(4000, es)
    grid = (es // be,)
    boff = ebase // be
    whole = es == e
    slice_spec = pl.BlockSpec((be, d), lambda i: (boff + i, 0))
    local_spec = pl.BlockSpec((be, d), lambda i: (i, 0))
    full = lambda a, b: pl.BlockSpec((a, b), lambda i: (0, 0))
    dst_in_specs = [] if whole else [
        pl.BlockSpec(memory_space=pltpu.MemorySpace.HBM)]
    dst_args = () if whole else (oute_in,)
    body = _edge_mlp_whole_kernel if whole else _edge_mlp_kernel
    return pl.pallas_call(
        body,
        grid=grid,
        in_specs=dst_in_specs + [
            slice_spec, local_spec, local_spec,
            full(d, h), full(1, h), full(h, h), full(1, h),
            full(h, d), full(1, d), full(1, d), full(1, d),
        ],
        out_specs=[local_spec, slice_spec],
        out_shape=[
            jax.ShapeDtypeStruct((es, d), jnp.float32),
            jax.ShapeDtypeStruct((e, d), jnp.float32),
        ],
        input_output_aliases={} if whole else {0: 1},
        compiler_params=pltpu.CompilerParams(
            dimension_semantics=("parallel",),
        ),
    )(*dst_args, h_edges, gs, gr, w1e, b1.reshape(1, -1), w2,
      b2.reshape(1, -1), w3, b3.reshape(1, -1), g.reshape(1, -1),
      bg.reshape(1, -1))


def _scatter_body(upd_hbm, rcv_hbm, zeros_hbm, out_hbm,
                  idx2d_v, rows0, rows1, sem0, sem1, acc_sh,
                  *, epw, chunk, nps, rem):
    cid = lax.axis_index("c")
    sid = lax.axis_index("s")
    wid = sid * _NC + cid
    base = wid * epw
    niters = epw // chunk
    npairs = niters // 2

    def _sliced_copy(src, dst):
        pltpu.sync_copy(src.at[pl.ds(sid * nps, nps)],
                        dst.at[pl.ds(sid * nps, nps)])
        if rem:
            @pl.when(sid == _NS - 1)
            def _():
                pltpu.sync_copy(src.at[pl.ds(_NS * nps, rem)],
                                dst.at[pl.ds(_NS * nps, rem)])

    _sliced_copy(zeros_hbm, acc_sh)
    pltpu.sync_copy(rcv_hbm.at[wid], idx2d_v)
    plsc.subcore_barrier()

    def l_start(i, rows, sem):
        pltpu.async_copy(upd_hbm.at[pl.ds(base + i * chunk, chunk)], rows,
                         sem)

    def l_wait(rows, sem):
        pltpu.make_async_copy(upd_hbm.at[pl.ds(base, chunk)], rows,
                              sem).wait()

    def add(i, rows):
        pltpu.sync_copy(rows, acc_sh.at[idx2d_v.at[i]], add=True)

    l_start(0, rows0, sem0)

    def pair(j, carry):
        i0 = 2 * j
        l_wait(rows0, sem0)
        l_start(i0 + 1, rows1, sem1)
        add(i0, rows0)
        l_wait(rows1, sem1)

        @pl.when(i0 + 2 < niters)
        def _():
            l_start(i0 + 2, rows0, sem0)

        add(i0 + 1, rows1)
        return carry

    lax.fori_loop(0, npairs, pair, 0, unroll=False)
    if niters % 2:
        l_wait(rows0, sem0)
        add(niters - 1, rows0)
    plsc.subcore_barrier()
    _sliced_copy(acc_sh, out_hbm.at[cid])


def _scatter(upd, receiver, n, ebase):
    e, h = upd.shape
    epw = e // _NW
    chunk = 80
    niters = epw // chunk
    nps = (n // _NS) // 8 * 8
    rem = n - _NS * nps
    zeros = jnp.zeros((n, h), jnp.float32)
    rcv3d = lax.dynamic_slice(receiver, (ebase,), (e,)).reshape(
        _NW, niters, chunk)
    kern = functools.partial(_scatter_body, epw=epw, chunk=chunk, nps=nps,
                             rem=rem)
    return pl.kernel(
        kern,
        mesh=_sc_mesh(),
        out_type=jax.ShapeDtypeStruct((_NC, n, h), jnp.float32),
        scratch_types=[
            pltpu.VMEM((niters, chunk), jnp.int32),
            pltpu.VMEM((chunk, h), jnp.float32),
            pltpu.VMEM((chunk, h), jnp.float32),
            pltpu.SemaphoreType.DMA,
            pltpu.SemaphoreType.DMA,
            pltpu.VMEM_SHARED((n, h), jnp.float32),
        ],
    )(upd, rcv3d, zeros)


def _node_mlp_kernel(*refs, nparts):
    hn_ref = refs[0]
    part_refs = refs[1:1 + nparts]
    (w1t_ref, w1b_ref, b1_ref, w2_ref, b2_ref, w3_ref, b3_ref, g_ref,
     bg_ref, out_ref) = refs[1 + nparts:]
    hn = hn_ref[...]
    agg = part_refs[0][...]
    for p_ref in part_refs[1:]:
        agg = agg + p_ref[...]
    x = (jnp.dot(hn, w1t_ref[...], preferred_element_type=jnp.float32)
         + jnp.dot(agg, w1b_ref[...], preferred_element_type=jnp.float32)
         + b1_ref[...])
    x = jnp.maximum(x, 0.0)
    x = jnp.dot(x, w2_ref[...], preferred_element_type=jnp.float32) + b2_ref[...]
    x = jnp.maximum(x, 0.0)
    x = jnp.dot(x, w3_ref[...], preferred_element_type=jnp.float32) + b3_ref[...]
    u = _layernorm_affine(x, g_ref[...], bg_ref[...])
    out_ref[...] = hn + u


def _node_mlp(h_nodes, parts, w1t, w1b, b1, w2, b2, w3, b3, g, bg):
    n, d = h_nodes.shape
    h = w1t.shape[1]
    bn = min(2000, n)
    grid = (n // bn,)
    nparts = len(parts)
    row_spec = pl.BlockSpec((bn, d), lambda i: (i, 0))
    full = lambda a, b: pl.BlockSpec((a, b), lambda i: (0, 0))
    return pl.pallas_call(
        functools.partial(_node_mlp_kernel, nparts=nparts),
        grid=grid,
        in_specs=[
            row_spec, *([row_spec] * nparts),
            full(d, h), full(d, h), full(1, h), full(h, h), full(1, h),
            full(h, d), full(1, d), full(1, d), full(1, d),
        ],
        out_specs=row_spec,
        out_shape=jax.ShapeDtypeStruct((n, d), jnp.float32),
        compiler_params=pltpu.CompilerParams(
            dimension_semantics=("parallel",),
        ),
    )(h_nodes, *parts, w1t, w1b, b1.reshape(1, -1),
      w2, b2.reshape(1, -1), w3, b3.reshape(1, -1),
      g.reshape(1, -1), bg.reshape(1, -1))


def kernel(h_nodes, h_edges, edge_index, We1, be1, We2, be2, We3, be3, ge,
           bge, Wn1, bn1, Wn2, bn2, Wn3, bn3, gn, bgn):
    n, d = h_nodes.shape
    e = h_edges.shape[0]
    sender = edge_index[0]
    receiver = edge_index[1]
    w1e, w1s, w1r = We1[:d], We1[d:2 * d], We1[2 * d:]

    nslices = 1
    es = e // nslices

    ps, pr = _proj(h_nodes, w1s, w1r)
    gathered = [_gather(ps, pr, sender, receiver, k * es, es)
                for k in range(nslices)]
    out_edges = None if nslices == 1 else jnp.zeros((e, d), jnp.float32)
    parts = []
    for k, (gs, gr) in enumerate(gathered):
        upd, out_edges = _edge_mlp(out_edges, h_edges, gs, gr, w1e, be1, We2,
                                   be2, We3, be3, ge, bge, k * es, es)
        pk = _scatter(upd, receiver, n, k * es)
        parts.extend([pk[0], pk[1]])
    out_nodes = _node_mlp(h_nodes, parts, Wn1[:d], Wn1[d:], bn1, Wn2, bn2,
                          Wn3, bn3, gn, bgn)
    return out_nodes, out_edges


# speedup vs baseline: 1.1251x; 1.0140x over previous
import functools

import jax
import jax.numpy as jnp
from jax import lax
from jax.experimental import pallas as pl
from jax.experimental.pallas import tpu as pltpu
from jax.experimental.pallas import tpu_sc as plsc

_NC = 2
_NS = 16
_NW = _NC * _NS

def _sc_mesh():
    return plsc.VectorSubcoreMesh(
        core_axis_name="c", subcore_axis_name="s",
        num_cores=_NC, num_subcores=_NS,
    )


def _layernorm_affine(x, g, b):
    mu = jnp.mean(x, axis=-1, keepdims=True)
    var = jnp.mean((x - mu) ** 2, axis=-1, keepdims=True)
    return (x - mu) * lax.rsqrt(var + 1e-5) * g + b


def _proj_kernel(x_ref, ws_ref, wr_ref, ps_ref, pr_ref):
    x = x_ref[...]
    ps_ref[...] = jnp.dot(x, ws_ref[...], preferred_element_type=jnp.float32)
    pr_ref[...] = jnp.dot(x, wr_ref[...], preferred_element_type=jnp.float32)


def _proj(h_nodes, w_s, w_r):
    n, d = h_nodes.shape
    h = w_s.shape[1]
    return pl.pallas_call(
        _proj_kernel,
        out_shape=[
            jax.ShapeDtypeStruct((n, h), jnp.float32),
            jax.ShapeDtypeStruct((n, h), jnp.float32),
        ],
    )(h_nodes, w_s, w_r)


def _gather_body(ps_hbm, pr_hbm, snd_hbm, rcv_hbm, gs_hbm, gr_hbm,
                 ids_v, idr_v, rs0, rr0, rs1, rr1,
                 gs0, gr0, gs1, gr1, ws0, wr0, ws1, wr1,
                 *, epw, chunk, ebase):
    wid = lax.axis_index("s") * _NC + lax.axis_index("c")
    base = wid * epw
    npairs = (epw // chunk) // 2

    pltpu.sync_copy(snd_hbm.at[pl.ds(ebase + base, epw)], ids_v)
    pltpu.sync_copy(rcv_hbm.at[pl.ds(ebase + base, epw)], idr_v)

    def g_start(i, rs, rr, sem_s, sem_r):
        pltpu.async_copy(ps_hbm.at[ids_v.at[pl.ds(i * chunk, chunk)]], rs,
                         sem_s)
        pltpu.async_copy(pr_hbm.at[idr_v.at[pl.ds(i * chunk, chunk)]], rr,
                         sem_r)

    def g_wait(rs, rr, sem_s, sem_r):
        pltpu.make_async_copy(ps_hbm.at[ids_v.at[pl.ds(0, chunk)]], rs,
                              sem_s).wait()
        pltpu.make_async_copy(pr_hbm.at[idr_v.at[pl.ds(0, chunk)]], rr,
                              sem_r).wait()

    def w_start(i, rs, rr, sem_s, sem_r):
        off = base + i * chunk
        pltpu.async_copy(rs, gs_hbm.at[pl.ds(off, chunk)], sem_s)
        pltpu.async_copy(rr, gr_hbm.at[pl.ds(off, chunk)], sem_r)

    def w_wait(rs, rr, sem_s, sem_r):
        pltpu.make_async_copy(rs, gs_hbm.at[pl.ds(base, chunk)], sem_s).wait()
        pltpu.make_async_copy(rr, gr_hbm.at[pl.ds(base, chunk)], sem_r).wait()

    g_start(0, rs0, rr0, gs0, gr0)

    def pair(j, carry):
        i0 = 2 * j
        g_wait(rs0, rr0, gs0, gr0)
        w_start(i0, rs0, rr0, ws0, wr0)

        @pl.when(j > 0)
        def _():
            w_wait(rs1, rr1, ws1, wr1)

        g_start(i0 + 1, rs1, rr1, gs1, gr1)
        g_wait(rs1, rr1, gs1, gr1)
        w_start(i0 + 1, rs1, rr1, ws1, wr1)
        w_wait(rs0, rr0, ws0, wr0)

        @pl.when(j + 1 < npairs)
        def _():
            g_start(i0 + 2, rs0, rr0, gs0, gr0)

        return carry

    lax.fori_loop(0, npairs, pair, 0, unroll=False)
    w_wait(rs1, rr1, ws1, wr1)


def _gather(ps, pr, sender, receiver, ebase, es):
    n, h = ps.shape
    epw = es // _NW
    chunk = 200
    kern = functools.partial(_gather_body, epw=epw, chunk=chunk, ebase=ebase)
    return pl.kernel(
        kern,
        mesh=_sc_mesh(),
        out_type=[
            jax.ShapeDtypeStruct((es, h), jnp.float32),
            jax.ShapeDtypeStruct((es, h), jnp.float32),
        ],
        scratch_types=[
            pltpu.VMEM((epw,), jnp.int32),
            pltpu.VMEM((epw,), jnp.int32),
            pltpu.VMEM((chunk, h), jnp.float32),
            pltpu.VMEM((chunk, h), jnp.float32),
            pltpu.VMEM((chunk, h), jnp.float32),
            pltpu.VMEM((chunk, h), jnp.float32),
        ] + [pltpu.SemaphoreType.DMA] * 8,
    )(ps, pr, sender, receiver)


def _edge_mlp_whole_kernel(*refs):
    _edge_mlp_kernel(None, *refs)


def _edge_mlp_kernel(dst_ref, he_ref, gs_ref, gr_ref, w1_ref, b1_ref, w2_ref,
                     b2_ref, w3_ref, b3_ref, g_ref, bg_ref, upd_ref, oute_ref):
    del dst_ref
    he = he_ref[...]
    x = jnp.dot(he, w1_ref[...], preferred_element_type=jnp.float32)
    x = x + gs_ref[...] + gr_ref[...] + b1_ref[...]
    x = jnp.maximum(x, 0.0)
    x = jnp.dot(x, w2_ref[...], preferred_element_type=jnp.float32) + b2_ref[...]
    x = jnp.maximum(x, 0.0)
    x = jnp.dot(x, w3_ref[...], preferred_element_type=jnp.float32) + b3_ref[...]
    u = _layernorm_affine(x, g_ref[...], bg_ref[...])
    upd_ref[...] = u
    oute_ref[...] = he + u


def _edge_mlp(oute_in, h_edges, gs, gr, w1e, b1, w2, b2, w3, b3, g, bg,
              ebase, es):
    e, d = h_edges.shape
    h = w1e.shape[1]
    be = min(8000, es)
    grid = (es // be,)
    boff = ebase // be
    whole = es == e
    slice_spec = pl.BlockSpec((be, d), lambda i: (boff + i, 0))
    local_spec = pl.BlockSpec((be, d), lambda i: (i, 0))
    full = lambda a, b: pl.BlockSpec((a, b), lambda i: (0, 0))
    dst_in_specs = [] if whole else [
        pl.BlockSpec(memory_space=pltpu.MemorySpace.HBM)]
    dst_args = () if whole else (oute_in,)
    body = _edge_mlp_whole_kernel if whole else _edge_mlp_kernel
    return pl.pallas_call(
        body,
        grid=grid,
        in_specs=dst_in_specs + [
            slice_spec, local_spec, local_spec,
            full(d, h), full(1, h), full(h, h), full(1, h),
            full(h, d), full(1, d), full(1, d), full(1, d),
        ],
        out_specs=[local_spec, slice_spec],
        out_shape=[
            jax.ShapeDtypeStruct((es, d), jnp.float32),
            jax.ShapeDtypeStruct((e, d), jnp.float32),
        ],
        input_output_aliases={} if whole else {0: 1},
        compiler_params=pltpu.CompilerParams(
            dimension_semantics=("parallel",),
        ),
    )(*dst_args, h_edges, gs, gr, w1e, b1.reshape(1, -1), w2,
      b2.reshape(1, -1), w3, b3.reshape(1, -1), g.reshape(1, -1),
      bg.reshape(1, -1))


def _scatter_body(upd_hbm, rcv_hbm, zeros_hbm, out_hbm,
                  idx2d_v, rows0, rows1, sem0, sem1, acc_sh,
                  *, epw, chunk, nps, rem):
    cid = lax.axis_index("c")
    sid = lax.axis_index("s")
    wid = sid * _NC + cid
    base = wid * epw
    niters = epw // chunk
    npairs = niters // 2

    def _sliced_copy(src, dst):
        pltpu.sync_copy(src.at[pl.ds(sid * nps, nps)],
                        dst.at[pl.ds(sid * nps, nps)])
        if rem:
            @pl.when(sid == _NS - 1)
            def _():
                pltpu.sync_copy(src.at[pl.ds(_NS * nps, rem)],
                                dst.at[pl.ds(_NS * nps, rem)])

    _sliced_copy(zeros_hbm, acc_sh)
    pltpu.sync_copy(rcv_hbm.at[wid], idx2d_v)
    plsc.subcore_barrier()

    def l_start(i, rows, sem):
        pltpu.async_copy(upd_hbm.at[pl.ds(base + i * chunk, chunk)], rows,
                         sem)

    def l_wait(rows, sem):
        pltpu.make_async_copy(upd_hbm.at[pl.ds(base, chunk)], rows,
                              sem).wait()

    def add(i, rows):
        pltpu.sync_copy(rows, acc_sh.at[idx2d_v.at[i]], add=True)

    l_start(0, rows0, sem0)

    def pair(j, carry):
        i0 = 2 * j
        l_wait(rows0, sem0)
        l_start(i0 + 1, rows1, sem1)
        add(i0, rows0)
        l_wait(rows1, sem1)

        @pl.when(i0 + 2 < niters)
        def _():
            l_start(i0 + 2, rows0, sem0)

        add(i0 + 1, rows1)
        return carry

    lax.fori_loop(0, npairs, pair, 0, unroll=False)
    if niters % 2:
        l_wait(rows0, sem0)
        add(niters - 1, rows0)
    plsc.subcore_barrier()
    _sliced_copy(acc_sh, out_hbm.at[cid])


def _scatter(upd, receiver, n, ebase):
    e, h = upd.shape
    epw = e // _NW
    chunk = 80
    niters = epw // chunk
    nps = (n // _NS) // 8 * 8
    rem = n - _NS * nps
    zeros = jnp.zeros((n, h), jnp.float32)
    rcv3d = lax.dynamic_slice(receiver, (ebase,), (e,)).reshape(
        _NW, niters, chunk)
    kern = functools.partial(_scatter_body, epw=epw, chunk=chunk, nps=nps,
                             rem=rem)
    return pl.kernel(
        kern,
        mesh=_sc_mesh(),
        out_type=jax.ShapeDtypeStruct((_NC, n, h), jnp.float32),
        scratch_types=[
            pltpu.VMEM((niters, chunk), jnp.int32),
            pltpu.VMEM((chunk, h), jnp.float32),
            pltpu.VMEM((chunk, h), jnp.float32),
            pltpu.SemaphoreType.DMA,
            pltpu.SemaphoreType.DMA,
            pltpu.VMEM_SHARED((n, h), jnp.float32),
        ],
    )(upd, rcv3d, zeros)


def _node_mlp_kernel(*refs, nparts):
    hn_ref = refs[0]
    part_refs = refs[1:1 + nparts]
    (w1t_ref, w1b_ref, b1_ref, w2_ref, b2_ref, w3_ref, b3_ref, g_ref,
     bg_ref, out_ref) = refs[1 + nparts:]
    hn = hn_ref[...]
    agg = part_refs[0][...]
    for p_ref in part_refs[1:]:
        agg = agg + p_ref[...]
    x = (jnp.dot(hn, w1t_ref[...], preferred_element_type=jnp.float32)
         + jnp.dot(agg, w1b_ref[...], preferred_element_type=jnp.float32)
         + b1_ref[...])
    x = jnp.maximum(x, 0.0)
    x = jnp.dot(x, w2_ref[...], preferred_element_type=jnp.float32) + b2_ref[...]
    x = jnp.maximum(x, 0.0)
    x = jnp.dot(x, w3_ref[...], preferred_element_type=jnp.float32) + b3_ref[...]
    u = _layernorm_affine(x, g_ref[...], bg_ref[...])
    out_ref[...] = hn + u


def _node_mlp(h_nodes, parts, w1t, w1b, b1, w2, b2, w3, b3, g, bg):
    n, d = h_nodes.shape
    h = w1t.shape[1]
    bn = min(2000, n)
    grid = (n // bn,)
    nparts = len(parts)
    row_spec = pl.BlockSpec((bn, d), lambda i: (i, 0))
    full = lambda a, b: pl.BlockSpec((a, b), lambda i: (0, 0))
    return pl.pallas_call(
        functools.partial(_node_mlp_kernel, nparts=nparts),
        grid=grid,
        in_specs=[
            row_spec, *([row_spec] * nparts),
            full(d, h), full(d, h), full(1, h), full(h, h), full(1, h),
            full(h, d), full(1, d), full(1, d), full(1, d),
        ],
        out_specs=row_spec,
        out_shape=jax.ShapeDtypeStruct((n, d), jnp.float32),
        compiler_params=pltpu.CompilerParams(
            dimension_semantics=("parallel",),
        ),
    )(h_nodes, *parts, w1t, w1b, b1.reshape(1, -1),
      w2, b2.reshape(1, -1), w3, b3.reshape(1, -1),
      g.reshape(1, -1), bg.reshape(1, -1))


def kernel(h_nodes, h_edges, edge_index, We1, be1, We2, be2, We3, be3, ge,
           bge, Wn1, bn1, Wn2, bn2, Wn3, bn3, gn, bgn):
    n, d = h_nodes.shape
    e = h_edges.shape[0]
    sender = edge_index[0]
    receiver = edge_index[1]
    w1e, w1s, w1r = We1[:d], We1[d:2 * d], We1[2 * d:]

    nslices = 1
    es = e // nslices

    ps, pr = _proj(h_nodes, w1s, w1r)
    gathered = [_gather(ps, pr, sender, receiver, k * es, es)
                for k in range(nslices)]
    out_edges = None if nslices == 1 else jnp.zeros((e, d), jnp.float32)
    parts = []
    for k, (gs, gr) in enumerate(gathered):
        upd, out_edges = _edge_mlp(out_edges, h_edges, gs, gr, w1e, be1, We2,
                                   be2, We3, be3, ge, bge, k * es, es)
        pk = _scatter(upd, receiver, n, k * es)
        parts.extend([pk[0], pk[1]])
    out_nodes = _node_mlp(h_nodes, parts, Wn1[:d], Wn1[d:], bn1, Wn2, bn2,
                          Wn3, bn3, gn, bgn)
    return out_nodes, out_edges


# speedup vs baseline: 1.3173x; 1.1708x over previous
import functools

import jax
import jax.numpy as jnp
from jax import lax
from jax.experimental import pallas as pl
from jax.experimental.pallas import tpu as pltpu
from jax.experimental.pallas import tpu_sc as plsc

_NC = 2
_NS = 16
_NW = _NC * _NS

def _sc_mesh():
    return plsc.VectorSubcoreMesh(
        core_axis_name="c", subcore_axis_name="s",
        num_cores=_NC, num_subcores=_NS,
    )


def _layernorm_affine(x, g, b):
    mu = jnp.mean(x, axis=-1, keepdims=True)
    var = jnp.mean((x - mu) ** 2, axis=-1, keepdims=True)
    return (x - mu) * lax.rsqrt(var + 1e-5) * g + b


def _proj_kernel(x_ref, ws_ref, wr_ref, ps_ref, pr_ref):
    x = x_ref[...]
    ps_ref[...] = jnp.dot(x, ws_ref[...], preferred_element_type=jnp.float32)
    pr_ref[...] = jnp.dot(x, wr_ref[...], preferred_element_type=jnp.float32)


def _proj(h_nodes, w_s, w_r):
    n, d = h_nodes.shape
    h = w_s.shape[1]
    return pl.pallas_call(
        _proj_kernel,
        out_shape=[
            jax.ShapeDtypeStruct((n, h), jnp.float32),
            jax.ShapeDtypeStruct((n, h), jnp.float32),
        ],
    )(h_nodes, w_s, w_r)


def _gather_body(ps_hbm, pr_hbm, snd_hbm, rcv_hbm, g_hbm,
                 ids_v, idr_v, r0, r1,
                 ga0, gb0, ga1, gb1, w0, w1,
                 *, epw, chunk, ebase):
    wid = lax.axis_index("s") * _NC + lax.axis_index("c")
    base = wid * epw
    npairs = (epw // chunk) // 2

    pltpu.sync_copy(snd_hbm.at[pl.ds(ebase + base, epw)], ids_v)
    pltpu.sync_copy(rcv_hbm.at[pl.ds(ebase + base, epw)], idr_v)

    def a_start(i, rows, sem):
        pltpu.async_copy(ps_hbm.at[ids_v.at[pl.ds(i * chunk, chunk)]], rows,
                         sem)

    def b_start(i, rows, sem):
        pltpu.async_copy(pr_hbm.at[idr_v.at[pl.ds(i * chunk, chunk)]], rows,
                         sem, add=True)

    def g_wait(rows, sem):
        pltpu.make_async_copy(ps_hbm.at[ids_v.at[pl.ds(0, chunk)]], rows,
                              sem).wait()

    def w_start(i, rows, sem):
        pltpu.async_copy(rows, g_hbm.at[pl.ds(base + i * chunk, chunk)], sem)

    def w_wait(rows, sem):
        pltpu.make_async_copy(rows, g_hbm.at[pl.ds(base, chunk)], sem).wait()

    a_start(0, r0, ga0)

    def pair(j, carry):
        i0 = 2 * j
        g_wait(r0, ga0)
        b_start(i0, r0, gb0)

        @pl.when(j > 0)
        def _():
            w_wait(r1, w1)

        a_start(i0 + 1, r1, ga1)
        g_wait(r0, gb0)
        w_start(i0, r0, w0)
        g_wait(r1, ga1)
        b_start(i0 + 1, r1, gb1)
        w_wait(r0, w0)

        @pl.when(j + 1 < npairs)
        def _():
            a_start(i0 + 2, r0, ga0)

        g_wait(r1, gb1)
        w_start(i0 + 1, r1, w1)
        return carry

    lax.fori_loop(0, npairs, pair, 0, unroll=False)
    w_wait(r1, w1)


def _gather(ps, pr, sender, receiver, ebase, es):
    n, h = ps.shape
    epw = es // _NW
    chunk = 200
    kern = functools.partial(_gather_body, epw=epw, chunk=chunk, ebase=ebase)
    return pl.kernel(
        kern,
        mesh=_sc_mesh(),
        out_type=jax.ShapeDtypeStruct((es, h), jnp.float32),
        scratch_types=[
            pltpu.VMEM((epw,), jnp.int32),
            pltpu.VMEM((epw,), jnp.int32),
            pltpu.VMEM((chunk, h), jnp.float32),
            pltpu.VMEM((chunk, h), jnp.float32),
        ] + [pltpu.SemaphoreType.DMA] * 6,
    )(ps, pr, sender, receiver)


def _edge_mlp_whole_kernel(*refs):
    _edge_mlp_kernel(None, *refs)


def _edge_mlp_kernel(dst_ref, he_ref, gsum_ref, w1_ref, b1_ref, w2_ref,
                     b2_ref, w3_ref, b3_ref, g_ref, bg_ref, upd_ref, oute_ref):
    del dst_ref
    he = he_ref[...]
    x = jnp.dot(he, w1_ref[...], preferred_element_type=jnp.float32)
    x = x + gsum_ref[...] + b1_ref[...]
    x = jnp.maximum(x, 0.0)
    x = jnp.dot(x, w2_ref[...], preferred_element_type=jnp.float32) + b2_ref[...]
    x = jnp.maximum(x, 0.0)
    x = jnp.dot(x, w3_ref[...], preferred_element_type=jnp.float32) + b3_ref[...]
    u = _layernorm_affine(x, g_ref[...], bg_ref[...])
    upd_ref[...] = u
    oute_ref[...] = he + u


def _edge_mlp(oute_in, h_edges, gsum, w1e, b1, w2, b2, w3, b3, g, bg,
              ebase, es):
    e, d = h_edges.shape
    h = w1e.shape[1]
    be = min(8000, es)
    grid = (es // be,)
    boff = ebase // be
    whole = es == e
    slice_spec = pl.BlockSpec((be, d), lambda i: (boff + i, 0))
    local_spec = pl.BlockSpec((be, d), lambda i: (i, 0))
    full = lambda a, b: pl.BlockSpec((a, b), lambda i: (0, 0))
    dst_in_specs = [] if whole else [
        pl.BlockSpec(memory_space=pltpu.MemorySpace.HBM)]
    dst_args = () if whole else (oute_in,)
    body = _edge_mlp_whole_kernel if whole else _edge_mlp_kernel
    return pl.pallas_call(
        body,
        grid=grid,
        in_specs=dst_in_specs + [
            slice_spec, local_spec,
            full(d, h), full(1, h), full(h, h), full(1, h),
            full(h, d), full(1, d), full(1, d), full(1, d),
        ],
        out_specs=[local_spec, slice_spec],
        out_shape=[
            jax.ShapeDtypeStruct((es, d), jnp.float32),
            jax.ShapeDtypeStruct((e, d), jnp.float32),
        ],
        input_output_aliases={} if whole else {0: 1},
        compiler_params=pltpu.CompilerParams(
            dimension_semantics=("parallel",),
        ),
    )(*dst_args, h_edges, gsum, w1e, b1.reshape(1, -1), w2,
      b2.reshape(1, -1), w3, b3.reshape(1, -1), g.reshape(1, -1),
      bg.reshape(1, -1))


def _scatter_body(upd_hbm, rcv_hbm, zeros_hbm, out_hbm,
                  idx2d_v, rows0, rows1, sem0, sem1, acc_sh,
                  *, epw, chunk, nps, rem):
    cid = lax.axis_index("c")
    sid = lax.axis_index("s")
    wid = sid * _NC + cid
    base = wid * epw
    niters = epw // chunk
    npairs = niters // 2

    def _sliced_copy(src, dst):
        pltpu.sync_copy(src.at[pl.ds(sid * nps, nps)],
                        dst.at[pl.ds(sid * nps, nps)])
        if rem:
            @pl.when(sid == _NS - 1)
            def _():
                pltpu.sync_copy(src.at[pl.ds(_NS * nps, rem)],
                                dst.at[pl.ds(_NS * nps, rem)])

    _sliced_copy(zeros_hbm, acc_sh)
    pltpu.sync_copy(rcv_hbm.at[wid], idx2d_v)
    plsc.subcore_barrier()

    def l_start(i, rows, sem):
        pltpu.async_copy(upd_hbm.at[pl.ds(base + i * chunk, chunk)], rows,
                         sem)

    def l_wait(rows, sem):
        pltpu.make_async_copy(upd_hbm.at[pl.ds(base, chunk)], rows,
                              sem).wait()

    def add(i, rows):
        pltpu.sync_copy(rows, acc_sh.at[idx2d_v.at[i]], add=True)

    l_start(0, rows0, sem0)

    def pair(j, carry):
        i0 = 2 * j
        l_wait(rows0, sem0)
        l_start(i0 + 1, rows1, sem1)
        add(i0, rows0)
        l_wait(rows1, sem1)

        @pl.when(i0 + 2 < niters)
        def _():
            l_start(i0 + 2, rows0, sem0)

        add(i0 + 1, rows1)
        return carry

    lax.fori_loop(0, npairs, pair, 0, unroll=False)
    if niters % 2:
        l_wait(rows0, sem0)
        add(niters - 1, rows0)
    plsc.subcore_barrier()
    _sliced_copy(acc_sh, out_hbm.at[cid])


def _scatter(upd, receiver, n, ebase):
    e, h = upd.shape
    epw = e // _NW
    chunk = 80
    niters = epw // chunk
    nps = (n // _NS) // 8 * 8
    rem = n - _NS * nps
    zeros = jnp.zeros((n, h), jnp.float32)
    rcv3d = lax.dynamic_slice(receiver, (ebase,), (e,)).reshape(
        _NW, niters, chunk)
    kern = functools.partial(_scatter_body, epw=epw, chunk=chunk, nps=nps,
                             rem=rem)
    return pl.kernel(
        kern,
        mesh=_sc_mesh(),
        out_type=jax.ShapeDtypeStruct((_NC, n, h), jnp.float32),
        scratch_types=[
            pltpu.VMEM((niters, chunk), jnp.int32),
            pltpu.VMEM((chunk, h), jnp.float32),
            pltpu.VMEM((chunk, h), jnp.float32),
            pltpu.SemaphoreType.DMA,
            pltpu.SemaphoreType.DMA,
            pltpu.VMEM_SHARED((n, h), jnp.float32),
        ],
    )(upd, rcv3d, zeros)


def _node_mlp_kernel(*refs, nparts):
    hn_ref = refs[0]
    part_refs = refs[1:1 + nparts]
    (w1t_ref, w1b_ref, b1_ref, w2_ref, b2_ref, w3_ref, b3_ref, g_ref,
     bg_ref, out_ref) = refs[1 + nparts:]
    hn = hn_ref[...]
    agg = part_refs[0][...]
    for p_ref in part_refs[1:]:
        agg = agg + p_ref[...]
    x = (jnp.dot(hn, w1t_ref[...], preferred_element_type=jnp.float32)
         + jnp.dot(agg, w1b_ref[...], preferred_element_type=jnp.float32)
         + b1_ref[...])
    x = jnp.maximum(x, 0.0)
    x = jnp.dot(x, w2_ref[...], preferred_element_type=jnp.float32) + b2_ref[...]
    x = jnp.maximum(x, 0.0)
    x = jnp.dot(x, w3_ref[...], preferred_element_type=jnp.float32) + b3_ref[...]
    u = _layernorm_affine(x, g_ref[...], bg_ref[...])
    out_ref[...] = hn + u


def _node_mlp(h_nodes, parts, w1t, w1b, b1, w2, b2, w3, b3, g, bg):
    n, d = h_nodes.shape
    h = w1t.shape[1]
    bn = min(2000, n)
    grid = (n // bn,)
    nparts = len(parts)
    row_spec = pl.BlockSpec((bn, d), lambda i: (i, 0))
    full = lambda a, b: pl.BlockSpec((a, b), lambda i: (0, 0))
    return pl.pallas_call(
        functools.partial(_node_mlp_kernel, nparts=nparts),
        grid=grid,
        in_specs=[
            row_spec, *([row_spec] * nparts),
            full(d, h), full(d, h), full(1, h), full(h, h), full(1, h),
            full(h, d), full(1, d), full(1, d), full(1, d),
        ],
        out_specs=row_spec,
        out_shape=jax.ShapeDtypeStruct((n, d), jnp.float32),
        compiler_params=pltpu.CompilerParams(
            dimension_semantics=("parallel",),
        ),
    )(h_nodes, *parts, w1t, w1b, b1.reshape(1, -1),
      w2, b2.reshape(1, -1), w3, b3.reshape(1, -1),
      g.reshape(1, -1), bg.reshape(1, -1))


def kernel(h_nodes, h_edges, edge_index, We1, be1, We2, be2, We3, be3, ge,
           bge, Wn1, bn1, Wn2, bn2, Wn3, bn3, gn, bgn):
    n, d = h_nodes.shape
    e = h_edges.shape[0]
    sender = edge_index[0]
    receiver = edge_index[1]
    w1e, w1s, w1r = We1[:d], We1[d:2 * d], We1[2 * d:]

    nslices = 1
    es = e // nslices

    ps, pr = _proj(h_nodes, w1s, w1r)
    gathered = [_gather(ps, pr, sender, receiver, k * es, es)
                for k in range(nslices)]
    out_edges = None if nslices == 1 else jnp.zeros((e, d), jnp.float32)
    parts = []
    for k, gsum in enumerate(gathered):
        upd, out_edges = _edge_mlp(out_edges, h_edges, gsum, w1e, be1, We2,
                                   be2, We3, be3, ge, bge, k * es, es)
        pk = _scatter(upd, receiver, n, k * es)
        parts.extend([pk[0], pk[1]])
    out_nodes = _node_mlp(h_nodes, parts, Wn1[:d], Wn1[d:], bn1, Wn2, bn2,
                          Wn3, bn3, gn, bgn)
    return out_nodes, out_edges


# speedup vs baseline: 1.3271x; 1.0074x over previous
import functools

import jax
import jax.numpy as jnp
from jax import lax
from jax.experimental import pallas as pl
from jax.experimental.pallas import tpu as pltpu
from jax.experimental.pallas import tpu_sc as plsc

_NC = 2
_NS = 16
_NW = _NC * _NS

def _sc_mesh():
    return plsc.VectorSubcoreMesh(
        core_axis_name="c", subcore_axis_name="s",
        num_cores=_NC, num_subcores=_NS,
    )


def _layernorm_affine(x, g, b):
    mu = jnp.mean(x, axis=-1, keepdims=True)
    var = jnp.mean((x - mu) ** 2, axis=-1, keepdims=True)
    return (x - mu) * lax.rsqrt(var + 1e-5) * g + b


def _proj_kernel(x_ref, ws_ref, wr_ref, ps_ref, pr_ref):
    x = x_ref[...]
    ps_ref[...] = jnp.dot(x, ws_ref[...], preferred_element_type=jnp.float32)
    pr_ref[...] = jnp.dot(x, wr_ref[...], preferred_element_type=jnp.float32)


def _proj(h_nodes, w_s, w_r):
    n, d = h_nodes.shape
    h = w_s.shape[1]
    return pl.pallas_call(
        _proj_kernel,
        out_shape=[
            jax.ShapeDtypeStruct((n, h), jnp.float32),
            jax.ShapeDtypeStruct((n, h), jnp.float32),
        ],
    )(h_nodes, w_s, w_r)


def _gather_body(ps_hbm, pr_hbm, snd_hbm, rcv_hbm, g_hbm,
                 ids_v, idr_v, r0, r1,
                 ga0, gb0, ga1, gb1, w0, w1,
                 *, epw, chunk, ebase):
    wid = lax.axis_index("s") * _NC + lax.axis_index("c")
    base = wid * epw
    npairs = (epw // chunk) // 2

    pltpu.sync_copy(snd_hbm.at[pl.ds(ebase + base, epw)], ids_v)
    pltpu.sync_copy(rcv_hbm.at[pl.ds(ebase + base, epw)], idr_v)

    def a_start(i, rows, sem):
        pltpu.async_copy(ps_hbm.at[ids_v.at[pl.ds(i * chunk, chunk)]], rows,
                         sem)

    def b_start(i, rows, sem):
        pltpu.async_copy(pr_hbm.at[idr_v.at[pl.ds(i * chunk, chunk)]], rows,
                         sem, add=True)

    def g_wait(rows, sem):
        pltpu.make_async_copy(ps_hbm.at[ids_v.at[pl.ds(0, chunk)]], rows,
                              sem).wait()

    def w_start(i, rows, sem):
        pltpu.async_copy(rows, g_hbm.at[pl.ds(base + i * chunk, chunk)], sem)

    def w_wait(rows, sem):
        pltpu.make_async_copy(rows, g_hbm.at[pl.ds(base, chunk)], sem).wait()

    a_start(0, r0, ga0)

    def pair(j, carry):
        i0 = 2 * j
        g_wait(r0, ga0)
        b_start(i0, r0, gb0)

        @pl.when(j > 0)
        def _():
            w_wait(r1, w1)

        a_start(i0 + 1, r1, ga1)
        g_wait(r0, gb0)
        w_start(i0, r0, w0)
        g_wait(r1, ga1)
        b_start(i0 + 1, r1, gb1)
        w_wait(r0, w0)

        @pl.when(j + 1 < npairs)
        def _():
            a_start(i0 + 2, r0, ga0)

        g_wait(r1, gb1)
        w_start(i0 + 1, r1, w1)
        return carry

    lax.fori_loop(0, npairs, pair, 0, unroll=False)
    w_wait(r1, w1)


def _gather(ps, pr, sender, receiver, ebase, es):
    n, h = ps.shape
    epw = es // _NW
    chunk = 400
    kern = functools.partial(_gather_body, epw=epw, chunk=chunk, ebase=ebase)
    return pl.kernel(
        kern,
        mesh=_sc_mesh(),
        out_type=jax.ShapeDtypeStruct((es, h), jnp.float32),
        scratch_types=[
            pltpu.VMEM((epw,), jnp.int32),
            pltpu.VMEM((epw,), jnp.int32),
            pltpu.VMEM((chunk, h), jnp.float32),
            pltpu.VMEM((chunk, h), jnp.float32),
        ] + [pltpu.SemaphoreType.DMA] * 6,
    )(ps, pr, sender, receiver)


def _edge_mlp_whole_kernel(*refs):
    _edge_mlp_kernel(None, *refs)


def _edge_mlp_kernel(dst_ref, he_ref, gsum_ref, w1_ref, b1_ref, w2_ref,
                     b2_ref, w3_ref, b3_ref, g_ref, bg_ref, upd_ref, oute_ref):
    del dst_ref
    he = he_ref[...]
    x = jnp.dot(he, w1_ref[...], preferred_element_type=jnp.float32)
    x = x + gsum_ref[...] + b1_ref[...]
    x = jnp.maximum(x, 0.0)
    x = jnp.dot(x, w2_ref[...], preferred_element_type=jnp.float32) + b2_ref[...]
    x = jnp.maximum(x, 0.0)
    x = jnp.dot(x, w3_ref[...], preferred_element_type=jnp.float32) + b3_ref[...]
    u = _layernorm_affine(x, g_ref[...], bg_ref[...])
    upd_ref[...] = u
    oute_ref[...] = he + u


def _edge_mlp(oute_in, h_edges, gsum, w1e, b1, w2, b2, w3, b3, g, bg,
              ebase, es):
    e, d = h_edges.shape
    h = w1e.shape[1]
    be = min(8000, es)
    grid = (es // be,)
    boff = ebase // be
    whole = es == e
    slice_spec = pl.BlockSpec((be, d), lambda i: (boff + i, 0))
    local_spec = pl.BlockSpec((be, d), lambda i: (i, 0))
    full = lambda a, b: pl.BlockSpec((a, b), lambda i: (0, 0))
    dst_in_specs = [] if whole else [
        pl.BlockSpec(memory_space=pltpu.MemorySpace.HBM)]
    dst_args = () if whole else (oute_in,)
    body = _edge_mlp_whole_kernel if whole else _edge_mlp_kernel
    return pl.pallas_call(
        body,
        grid=grid,
        in_specs=dst_in_specs + [
            slice_spec, local_spec,
            full(d, h), full(1, h), full(h, h), full(1, h),
            full(h, d), full(1, d), full(1, d), full(1, d),
        ],
        out_specs=[local_spec, slice_spec],
        out_shape=[
            jax.ShapeDtypeStruct((es, d), jnp.float32),
            jax.ShapeDtypeStruct((e, d), jnp.float32),
        ],
        input_output_aliases={} if whole else {0: 1},
        compiler_params=pltpu.CompilerParams(
            dimension_semantics=("parallel",),
        ),
    )(*dst_args, h_edges, gsum, w1e, b1.reshape(1, -1), w2,
      b2.reshape(1, -1), w3, b3.reshape(1, -1), g.reshape(1, -1),
      bg.reshape(1, -1))


def _scatter_body(upd_hbm, rcv_hbm, zeros_hbm, out_hbm,
                  idx2d_v, rows0, rows1, sem0, sem1, acc_sh,
                  *, epw, chunk, nps, rem):
    cid = lax.axis_index("c")
    sid = lax.axis_index("s")
    wid = sid * _NC + cid
    base = wid * epw
    niters = epw // chunk
    npairs = niters // 2

    def _sliced_copy(src, dst):
        pltpu.sync_copy(src.at[pl.ds(sid * nps, nps)],
                        dst.at[pl.ds(sid * nps, nps)])
        if rem:
            @pl.when(sid == _NS - 1)
            def _():
                pltpu.sync_copy(src.at[pl.ds(_NS * nps, rem)],
                                dst.at[pl.ds(_NS * nps, rem)])

    _sliced_copy(zeros_hbm, acc_sh)
    pltpu.sync_copy(rcv_hbm.at[wid], idx2d_v)
    plsc.subcore_barrier()

    def l_start(i, rows, sem):
        pltpu.async_copy(upd_hbm.at[pl.ds(base + i * chunk, chunk)], rows,
                         sem)

    def l_wait(rows, sem):
        pltpu.make_async_copy(upd_hbm.at[pl.ds(base, chunk)], rows,
                              sem).wait()

    def add(i, rows):
        pltpu.sync_copy(rows, acc_sh.at[idx2d_v.at[i]], add=True)

    l_start(0, rows0, sem0)

    def pair(j, carry):
        i0 = 2 * j
        l_wait(rows0, sem0)
        l_start(i0 + 1, rows1, sem1)
        add(i0, rows0)
        l_wait(rows1, sem1)

        @pl.when(i0 + 2 < niters)
        def _():
            l_start(i0 + 2, rows0, sem0)

        add(i0 + 1, rows1)
        return carry

    lax.fori_loop(0, npairs, pair, 0, unroll=False)
    if niters % 2:
        l_wait(rows0, sem0)
        add(niters - 1, rows0)
    plsc.subcore_barrier()
    _sliced_copy(acc_sh, out_hbm.at[cid])


def _scatter(upd, receiver, n, ebase):
    e, h = upd.shape
    epw = e // _NW
    chunk = 80
    niters = epw // chunk
    nps = (n // _NS) // 8 * 8
    rem = n - _NS * nps
    zeros = jnp.zeros((n, h), jnp.float32)
    rcv3d = lax.dynamic_slice(receiver, (ebase,), (e,)).reshape(
        _NW, niters, chunk)
    kern = functools.partial(_scatter_body, epw=epw, chunk=chunk, nps=nps,
                             rem=rem)
    return pl.kernel(
        kern,
        mesh=_sc_mesh(),
        out_type=jax.ShapeDtypeStruct((_NC, n, h), jnp.float32),
        scratch_types=[
            pltpu.VMEM((niters, chunk), jnp.int32),
            pltpu.VMEM((chunk, h), jnp.float32),
            pltpu.VMEM((chunk, h), jnp.float32),
            pltpu.SemaphoreType.DMA,
            pltpu.SemaphoreType.DMA,
            pltpu.VMEM_SHARED((n, h), jnp.float32),
        ],
    )(upd, rcv3d, zeros)


def _node_mlp_kernel(*refs, nparts):
    hn_ref = refs[0]
    part_refs = refs[1:1 + nparts]
    (w1t_ref, w1b_ref, b1_ref, w2_ref, b2_ref, w3_ref, b3_ref, g_ref,
     bg_ref, out_ref) = refs[1 + nparts:]
    hn = hn_ref[...]
    agg = part_refs[0][...]
    for p_ref in part_refs[1:]:
        agg = agg + p_ref[...]
    x = (jnp.dot(hn, w1t_ref[...], preferred_element_type=jnp.float32)
         + jnp.dot(agg, w1b_ref[...], preferred_element_type=jnp.float32)
         + b1_ref[...])
    x = jnp.maximum(x, 0.0)
    x = jnp.dot(x, w2_ref[...], preferred_element_type=jnp.float32) + b2_ref[...]
    x = jnp.maximum(x, 0.0)
    x = jnp.dot(x, w3_ref[...], preferred_element_type=jnp.float32) + b3_ref[...]
    u = _layernorm_affine(x, g_ref[...], bg_ref[...])
    out_ref[...] = hn + u


def _node_mlp(h_nodes, parts, w1t, w1b, b1, w2, b2, w3, b3, g, bg):
    n, d = h_nodes.shape
    h = w1t.shape[1]
    bn = min(2000, n)
    grid = (n // bn,)
    nparts = len(parts)
    row_spec = pl.BlockSpec((bn, d), lambda i: (i, 0))
    full = lambda a, b: pl.BlockSpec((a, b), lambda i: (0, 0))
    return pl.pallas_call(
        functools.partial(_node_mlp_kernel, nparts=nparts),
        grid=grid,
        in_specs=[
            row_spec, *([row_spec] * nparts),
            full(d, h), full(d, h), full(1, h), full(h, h), full(1, h),
            full(h, d), full(1, d), full(1, d), full(1, d),
        ],
        out_specs=row_spec,
        out_shape=jax.ShapeDtypeStruct((n, d), jnp.float32),
        compiler_params=pltpu.CompilerParams(
            dimension_semantics=("parallel",),
        ),
    )(h_nodes, *parts, w1t, w1b, b1.reshape(1, -1),
      w2, b2.reshape(1, -1), w3, b3.reshape(1, -1),
      g.reshape(1, -1), bg.reshape(1, -1))


def kernel(h_nodes, h_edges, edge_index, We1, be1, We2, be2, We3, be3, ge,
           bge, Wn1, bn1, Wn2, bn2, Wn3, bn3, gn, bgn):
    n, d = h_nodes.shape
    e = h_edges.shape[0]
    sender = edge_index[0]
    receiver = edge_index[1]
    w1e, w1s, w1r = We1[:d], We1[d:2 * d], We1[2 * d:]

    nslices = 1
    es = e // nslices

    ps, pr = _proj(h_nodes, w1s, w1r)
    gathered = [_gather(ps, pr, sender, receiver, k * es, es)
                for k in range(nslices)]
    out_edges = None if nslices == 1 else jnp.zeros((e, d), jnp.float32)
    parts = []
    for k, gsum in enumerate(gathered):
        upd, out_edges = _edge_mlp(out_edges, h_edges, gsum, w1e, be1, We2,
                                   be2, We3, be3, ge, bge, k * es, es)
        pk = _scatter(upd, receiver, n, k * es)
        parts.extend([pk[0], pk[1]])
    out_nodes = _node_mlp(h_nodes, parts, Wn1[:d], Wn1[d:], bn1, Wn2, bn2,
                          Wn3, bn3, gn, bgn)
    return out_nodes, out_edges
